# baseline (speedup 1.0000x reference)
import functools

import jax
import jax.numpy as jnp
from jax import lax
from jax.experimental import pallas as pl
from jax.experimental.pallas import tpu as pltpu
from jax.experimental.pallas import tpu_sc as plsc

_HIDDEN = 128
_EPS = 1e-12


def _make_sc_gather(n_tok):
    info = plsc.get_sparse_core_info()
    nc, ns = info.num_cores, info.num_subcores
    nw = nc * ns
    tpw = n_tok // nw
    n_chunks = tpw // 128
    mesh = plsc.VectorSubcoreMesh(core_axis_name="c", subcore_axis_name="s")

    @functools.partial(
        pl.kernel,
        out_type=jax.ShapeDtypeStruct((n_tok, _HIDDEN), jnp.float32),
        mesh=mesh,
        scratch_types=[
            pltpu.VMEM((n_chunks, 128), jnp.int32),
            pltpu.VMEM((tpw, _HIDDEN), jnp.float32),
            pltpu.SemaphoreType.DMA,
        ],
    )
    def sc_gather(ids_hbm, word_hbm, out_hbm, idx_v, rows_v, sem):
        wid = lax.axis_index("s") * nc + lax.axis_index("c")
        base = wid * tpw
        pltpu.sync_copy(ids_hbm.at[pl.ds(wid * n_chunks, n_chunks)], idx_v)
        copies = [
            pltpu.async_copy(word_hbm.at[idx_v.at[j]],
                             rows_v.at[pl.ds(j * 128, 128)], sem)
            for j in range(n_chunks)
        ]
        for c in copies:
            c.wait()
        pltpu.sync_copy(rows_v, out_hbm.at[pl.ds(base, tpw)])

    return sc_gather


def _tc_epilogue(rows, ttf, pos_emb, type_emb, ln_gamma, ln_beta, blk):
    n_tok = rows.shape[0]
    seq_len = pos_emb.shape[0]
    grid = (n_tok // blk,)
    pos_blocks = seq_len // blk

    def body(rows_ref, ttf_ref, pos_ref, type_ref, gam_ref, bet_ref, out_ref):
        t0 = type_ref[0:1, :]
        t1 = type_ref[1:2, :]
        x = rows_ref[...] + pos_ref[...] + t0 + ttf_ref[...] * (t1 - t0)
        mu = jnp.mean(x, axis=-1, keepdims=True)
        var = jnp.mean(jnp.square(x - mu), axis=-1, keepdims=True)
        out_ref[...] = ((x - mu) * lax.rsqrt(var + _EPS) * gam_ref[0:1, :]
                        + bet_ref[0:1, :])

    return pl.pallas_call(
        body,
        grid=grid,
        in_specs=[
            pl.BlockSpec((blk, _HIDDEN), lambda i: (i, 0)),
            pl.BlockSpec((blk, 1), lambda i: (i, 0)),
            pl.BlockSpec((blk, _HIDDEN), lambda i: (i % pos_blocks, 0)),
            pl.BlockSpec((2, _HIDDEN), lambda i: (0, 0)),
            pl.BlockSpec((1, _HIDDEN), lambda i: (0, 0)),
            pl.BlockSpec((1, _HIDDEN), lambda i: (0, 0)),
        ],
        out_specs=pl.BlockSpec((blk, _HIDDEN), lambda i: (i, 0)),
        out_shape=jax.ShapeDtypeStruct((n_tok, _HIDDEN), jnp.float32),
    )(rows, ttf, pos_emb, type_emb, ln_gamma, ln_beta)


@jax.jit
def kernel(input_ids, token_type_ids, word_emb, pos_emb, type_emb, ln_gamma,
           ln_beta):
    b, s = input_ids.shape
    n_tok = b * s
    ids = input_ids.astype(jnp.int32).reshape(n_tok // 128, 128)
    rows = _make_sc_gather(n_tok)(ids, word_emb)
    ttf = token_type_ids.reshape(n_tok, 1).astype(jnp.float32)
    out = _tc_epilogue(rows, ttf, pos_emb, type_emb,
                       ln_gamma.reshape(1, _HIDDEN),
                       ln_beta.reshape(1, _HIDDEN), blk=512)
    return out.reshape(b, s, _HIDDEN)
